# baseline (speedup 1.0000x reference)
import jax
import jax.numpy as jnp
from jax import lax
from jax.experimental import pallas as pl
from jax.experimental.pallas import tpu as pltpu
from jax.experimental.pallas import tpu_sc as plsc

_K = 32
_C = 384
_B = 16
_H = 56
_W = 56
_HW = _H * _W
_COLS_PER_W = _HW // 2
_P = 112
_NBLK = _COLS_PER_W // _P


def _sort_desc(x):
    return plsc.sort_key_val(x, x, descending=True)[0]


def _topk_body(x_hbm, out_hbm, buf0, buf1, outbuf, sem):
    cid = lax.axis_index("c")
    sid = lax.axis_index("s")
    b = sid
    col0 = cid * _COLS_PER_W
    last_lane = lax.iota(jnp.int32, 16) == 15

    def start(blk, buf):
        pltpu.make_async_copy(
            x_hbm.at[b, pl.ds((col0 + blk * _P) * _C, _P * _C)], buf, sem
        ).start()

    def wait(buf):
        pltpu.make_async_copy(
            x_hbm.at[b, pl.ds(col0 * _C, _P * _C)], buf, sem
        ).wait()

    def process(buf, blk):
        @plsc.parallel_loop(0, _P, 1, unroll=3)
        def _pos_loop(pos):
            base = pos * _C

            def load(k):
                return buf[pl.ds(base + 16 * k, 16)]

            s1d = _sort_desc(load(0))
            s2a = jnp.sort(load(1))
            p = jnp.maximum(s1d, s2a)
            q = jnp.minimum(s1d, s2a)
            t0 = _sort_desc(p)
            t1 = _sort_desc(q)
            nblk32 = _C // 32
            for k in range(1, nblk32):
                s1d = _sort_desc(load(2 * k))
                s2a = jnp.sort(load(2 * k + 1))
                p = jnp.maximum(s1d, s2a)
                q = jnp.minimum(s1d, s2a)
                pa = jnp.sort(p)
                qa = jnp.sort(q)
                w0 = jnp.maximum(t0, qa)
                w1 = jnp.maximum(t1, pa)
                if k < nblk32 - 1:
                    a = jnp.maximum(w0, w1)
                    bt = jnp.minimum(w0, w1)
                    t0 = _sort_desc(a)
                    t1 = _sort_desc(bt)
                else:
                    t0, t1 = w0, w1
            acc = plsc.cumsum(t0 + t1) * (1.0 / _K)
            plsc.store_scatter(
                outbuf,
                [jnp.full((16,), blk * _P + pos, jnp.int32)],
                acc,
                mask=last_lane,
            )

    start(0, buf0)

    def panel_pair(gg, _):
        for j, (buf_a, buf_b) in enumerate(((buf0, buf1), (buf1, buf0))):
            blk = gg * 2 + j
            wait(buf_a)
            nxt = blk + 1

            @pl.when(nxt < _NBLK)
            def _():
                start(nxt, buf_b)

            process(buf_a, blk)
        return 0

    lax.fori_loop(0, _NBLK // 2, panel_pair, 0)
    pltpu.sync_copy(outbuf, out_hbm.at[b, pl.ds(col0, _COLS_PER_W)])


def _make_kernel(interpret=False):
    return pl.kernel(
        _topk_body,
        out_type=jax.ShapeDtypeStruct((_B, _HW), jnp.float32),
        mesh=plsc.VectorSubcoreMesh(
            core_axis_name="c",
            subcore_axis_name="s",
            num_cores=2,
            num_subcores=16,
        ),
        scratch_types=[
            pltpu.VMEM((_P * _C,), jnp.float32),
            pltpu.VMEM((_P * _C,), jnp.float32),
            pltpu.VMEM((_COLS_PER_W,), jnp.float32),
            pltpu.SemaphoreType.DMA,
        ],
        compiler_params=pltpu.CompilerParams(
            use_tc_tiling_on_sc=False, needs_layout_passes=False
        ),
        interpret=interpret,
    )


@jax.jit
def kernel(input):
    x = input.reshape(_B, _C, _HW).transpose(0, 2, 1).reshape(_B, _HW * _C)
    out = _make_kernel()(x)
    return out.reshape(_B, 1, _H, _W)


# speedup vs baseline: 1.0569x; 1.0569x over previous
import jax
import jax.numpy as jnp
from jax import lax
from jax.experimental import pallas as pl
from jax.experimental.pallas import tpu as pltpu
from jax.experimental.pallas import tpu_sc as plsc

_K = 32
_C = 384
_B = 16
_H = 56
_W = 56
_HW = _H * _W
_COLS_PER_W = _HW // 2
_P = 112
_NBLK = _COLS_PER_W // _P


def _sort_desc(x):
    return plsc.sort_key_val(x, x, descending=True)[0]


def _topk_body(x_hbm, out_hbm, buf0, buf1, outbuf, sem):
    cid = lax.axis_index("c")
    sid = lax.axis_index("s")
    b = sid
    col0 = cid * _COLS_PER_W
    last_lane = lax.iota(jnp.int32, 16) == 15

    def start(blk, buf):
        pltpu.make_async_copy(
            x_hbm.at[b, pl.ds((col0 + blk * _P) * _C, _P * _C)], buf, sem
        ).start()

    def wait(buf):
        pltpu.make_async_copy(
            x_hbm.at[b, pl.ds(col0 * _C, _P * _C)], buf, sem
        ).wait()

    def process(buf, blk):
        @plsc.parallel_loop(0, _P, 1, unroll=2)
        def _pos_loop(pos):
            base = pos * _C

            def load(k):
                return buf[pl.ds(base + 16 * k, 16)]

            s1d = _sort_desc(load(0))
            s2a = jnp.sort(load(1))
            p = jnp.maximum(s1d, s2a)
            q = jnp.minimum(s1d, s2a)
            t0 = _sort_desc(p)
            t1 = _sort_desc(q)
            nblk32 = _C // 32
            for k in range(1, nblk32):
                s1d = _sort_desc(load(2 * k))
                s2a = jnp.sort(load(2 * k + 1))
                p = jnp.maximum(s1d, s2a)
                q = jnp.minimum(s1d, s2a)
                pa = jnp.sort(p)
                qa = jnp.sort(q)
                w0 = jnp.maximum(t0, qa)
                w1 = jnp.maximum(t1, pa)
                if k < nblk32 - 1:
                    a = jnp.maximum(w0, w1)
                    bt = jnp.minimum(w0, w1)
                    t0 = _sort_desc(a)
                    t1 = _sort_desc(bt)
                else:
                    t0, t1 = w0, w1
            acc = plsc.cumsum(t0 + t1) * (1.0 / _K)
            plsc.store_scatter(
                outbuf,
                [jnp.full((16,), blk * _P + pos, jnp.int32)],
                acc,
                mask=last_lane,
            )

    start(0, buf0)

    def panel_pair(gg, _):
        for j, (buf_a, buf_b) in enumerate(((buf0, buf1), (buf1, buf0))):
            blk = gg * 2 + j
            wait(buf_a)
            nxt = blk + 1

            @pl.when(nxt < _NBLK)
            def _():
                start(nxt, buf_b)

            process(buf_a, blk)
        return 0

    lax.fori_loop(0, _NBLK // 2, panel_pair, 0)
    pltpu.sync_copy(outbuf, out_hbm.at[b, pl.ds(col0, _COLS_PER_W)])


def _make_kernel(interpret=False):
    return pl.kernel(
        _topk_body,
        out_type=jax.ShapeDtypeStruct((_B, _HW), jnp.float32),
        mesh=plsc.VectorSubcoreMesh(
            core_axis_name="c",
            subcore_axis_name="s",
            num_cores=2,
            num_subcores=16,
        ),
        scratch_types=[
            pltpu.VMEM((_P * _C,), jnp.float32),
            pltpu.VMEM((_P * _C,), jnp.float32),
            pltpu.VMEM((_COLS_PER_W,), jnp.float32),
            pltpu.SemaphoreType.DMA,
        ],
        compiler_params=pltpu.CompilerParams(
            use_tc_tiling_on_sc=False, needs_layout_passes=False
        ),
        interpret=interpret,
    )


@jax.jit
def kernel(input):
    x = input.reshape(_B, _C, _HW).transpose(0, 2, 1).reshape(_B, _HW * _C)
    out = _make_kernel()(x)
    return out.reshape(_B, 1, _H, _W)
